# baseline (speedup 1.0000x reference)
import functools

import jax
import jax.numpy as jnp
from jax import lax
from jax.experimental import pallas as pl
from jax.experimental.pallas import tpu as pltpu
from jax.experimental.pallas import tpu_sc as plsc

NC = 2
NS = 16
LANES = 16
CHUNK = 128
SG = 40


def _mlp_v_body(x_ref, w1_ref, b1_ref, w2_ref, b2_ref, o_ref):
    h = jnp.maximum(
        jnp.dot(x_ref[...], w1_ref[...], preferred_element_type=jnp.float32)
        + b1_ref[...], 0.0)
    o_ref[...] = (
        jnp.dot(h, w2_ref[...], preferred_element_type=jnp.float32)
        + b2_ref[...])


def _mlp_v(xx, w1, b1, w2, b2, blk):
    rows, din = xx.shape
    dh = w1.shape[1]
    dout = w2.shape[1]
    return pl.pallas_call(
        _mlp_v_body,
        grid=(rows // blk,),
        in_specs=[
            pl.BlockSpec((blk, din), lambda i: (i, 0)),
            pl.BlockSpec((din, dh), lambda i: (0, 0)),
            pl.BlockSpec((1, dh), lambda i: (0, 0)),
            pl.BlockSpec((dh, dout), lambda i: (0, 0)),
            pl.BlockSpec((1, dout), lambda i: (0, 0)),
        ],
        out_specs=pl.BlockSpec((blk, dout), lambda i: (i, 0)),
        out_shape=jax.ShapeDtypeStruct((rows, dout), jnp.float32),
    )(xx, w1, b1.reshape(1, dh), w2, b2.reshape(1, dout))


def _mlp_c_body(xt_ref, w1_ref, b1_ref, w2_ref, b2_ref, o_ref):
    h = jnp.maximum(
        lax.dot_general(xt_ref[...], w1_ref[...],
                        (((0,), (0,)), ((), ())),
                        preferred_element_type=jnp.float32)
        + b1_ref[...], 0.0)
    o_ref[...] = (
        jnp.dot(h.astype(jnp.bfloat16), w2_ref[...],
                preferred_element_type=jnp.float32)
        + b2_ref[...])


def _mlp_c(xt, w1, b1, w2, b2, blk, out_rows):
    din, rows = xt.shape
    dh = w1.shape[1]
    dout = w2.shape[1]
    last_real = rows // blk - 1
    return pl.pallas_call(
        _mlp_c_body,
        grid=(out_rows // blk,),
        in_specs=[
            pl.BlockSpec((din, blk), lambda i: (0, jnp.minimum(i, last_real))),
            pl.BlockSpec((din, dh), lambda i: (0, 0)),
            pl.BlockSpec((1, dh), lambda i: (0, 0)),
            pl.BlockSpec((dh, dout), lambda i: (0, 0)),
            pl.BlockSpec((1, dout), lambda i: (0, 0)),
        ],
        out_specs=pl.BlockSpec((blk, dout), lambda i: (i, 0)),
        out_shape=jax.ShapeDtypeStruct((out_rows, dout), jnp.float32),
    )(xt, w1, b1.reshape(1, dh), w2.astype(jnp.bfloat16),
      b2.reshape(1, dout))


def _combine_body(a_ref, b_ref, c_ref, e_ref, o_ref):
    o_ref[...] = (a_ref[0] + b_ref[0]) + (c_ref[0] + e_ref[0])


def _combine(pv, pc, n, blk):
    _, n_acc, d = pv.shape
    return pl.pallas_call(
        _combine_body,
        grid=(n // blk,),
        in_specs=[
            pl.BlockSpec((1, blk, d), lambda i: (0, i, 0)),
            pl.BlockSpec((1, blk, d), lambda i: (1, i, 0)),
            pl.BlockSpec((1, blk, d), lambda i: (0, i, 0)),
            pl.BlockSpec((1, blk, d), lambda i: (1, i, 0)),
        ],
        out_specs=pl.BlockSpec((blk, d), lambda i: (i, 0)),
        out_shape=jax.ShapeDtypeStruct((n, d), jnp.float32),
    )(pv, pv, pc, pc)


def _sc_prologue(sid, buf0, acc_sh, stripe, d):
    def _zrow(r, carry):
        for j in range(d // LANES):
            buf0[r, pl.ds(j * LANES, LANES)] = jnp.zeros((LANES,), jnp.float32)
        return carry
    lax.fori_loop(0, CHUNK, _zrow, 0)
    for k in range(stripe // CHUNK):
        pltpu.sync_copy(buf0,
                        acc_sh.at[pl.ds(sid * stripe + k * CHUNK, CHUNK)])
    plsc.subcore_barrier()


def _sc_epilogue(cid, sid, buf0, acc_sh, out_hbm, stripe):
    plsc.subcore_barrier()
    for k in range(stripe // CHUNK):
        r0 = sid * stripe + k * CHUNK
        pltpu.sync_copy(acc_sh.at[pl.ds(r0, CHUNK)], buf0)
        pltpu.sync_copy(buf0, out_hbm.at[cid, pl.ds(r0, CHUNK)])


V_SPLIT = (160, 0)


def _sc_gather_scatter(hv, src2, dst2, n_acc):
    n_nodes, d = hv.shape
    stripe = n_acc // NS

    mesh = plsc.VectorSubcoreMesh(core_axis_name="c", subcore_axis_name="s")

    @functools.partial(
        pl.kernel,
        mesh=mesh,
        out_type=jax.ShapeDtypeStruct((NC, n_acc, d), jnp.float32),
        scratch_types=[
            pltpu.VMEM((SG, CHUNK), jnp.int32),
            pltpu.VMEM((SG, CHUNK), jnp.int32),
            pltpu.VMEM((CHUNK, d), jnp.float32),
            pltpu.VMEM((CHUNK, d), jnp.float32),
            pltpu.VMEM_SHARED((n_acc, d), jnp.float32),
            pltpu.SemaphoreType.DMA,
            pltpu.SemaphoreType.DMA,
        ],
    )
    def body(hv_hbm, src_hbm, dst_hbm, out_hbm,
             src_v, dst_v, buf0, buf1, acc_sh, sem0, sem1):
        cid = lax.axis_index("c")
        sid = lax.axis_index("s")
        bufs = ((buf0, sem0), (buf1, sem1))
        _sc_prologue(sid, buf0, acc_sh, stripe, d)

        def _run(base, count):
            stage_sizes = [SG] * (count // SG)
            if count % SG:
                stage_sizes.append(count % SG)
            off = 0
            for sg in stage_sizes:
                c0 = base + off
                off += sg
                pltpu.sync_copy(src_hbm.at[pl.ds(c0, sg)],
                                src_v.at[pl.ds(0, sg)])
                pltpu.sync_copy(dst_hbm.at[pl.ds(c0, sg)],
                                dst_v.at[pl.ds(0, sg)])
                pltpu.make_async_copy(
                    hv_hbm.at[src_v.at[0]], buf0, sem0).start()
                pltpu.make_async_copy(
                    hv_hbm.at[src_v.at[1]], buf1, sem1).start()

                def _pair(g, carry):
                    for b, (buf, sem) in enumerate(bufs):
                        ci = 2 * g + b
                        pltpu.make_async_copy(
                            hv_hbm.at[src_v.at[ci]], buf, sem).wait()
                        pltpu.sync_copy(buf, acc_sh.at[dst_v.at[ci]],
                                        add=True)

                        @pl.when(ci + 2 < sg)
                        def _():
                            pltpu.make_async_copy(
                                hv_hbm.at[src_v.at[ci + 2]], buf, sem).start()
                    return carry
                lax.fori_loop(0, sg // 2, _pair, 0)

        if V_SPLIT[0]:
            @pl.when(cid == 0)
            def _core0():
                _run(sid * V_SPLIT[0], V_SPLIT[0])

        if V_SPLIT[1]:
            @pl.when(cid == 1)
            def _core1():
                _run(NS * V_SPLIT[0] + sid * V_SPLIT[1], V_SPLIT[1])

        _sc_epilogue(cid, sid, buf0, acc_sh, out_hbm, stripe)

    return body(hv, src2, dst2)


def _sc_linear_scatter(hc, dst2, n_acc):
    d = hc.shape[1]
    per_tile = dst2.shape[0] // (NC * NS)
    n_stages = per_tile // SG
    stripe = n_acc // NS

    mesh = plsc.VectorSubcoreMesh(core_axis_name="c", subcore_axis_name="s")

    @functools.partial(
        pl.kernel,
        mesh=mesh,
        out_type=jax.ShapeDtypeStruct((NC, n_acc, d), jnp.float32),
        scratch_types=[
            pltpu.VMEM((SG, CHUNK), jnp.int32),
            pltpu.VMEM((CHUNK, d), jnp.float32),
            pltpu.VMEM((CHUNK, d), jnp.float32),
            pltpu.VMEM_SHARED((n_acc, d), jnp.float32),
            pltpu.SemaphoreType.DMA,
            pltpu.SemaphoreType.DMA,
        ],
    )
    def body(hc_hbm, dst_hbm, out_hbm,
             dst_v, buf0, buf1, acc_sh, sem0, sem1):
        cid = lax.axis_index("c")
        sid = lax.axis_index("s")
        bufs = ((buf0, sem0), (buf1, sem1))
        _sc_prologue(sid, buf0, acc_sh, stripe, d)

        for s in range(n_stages):
            c0 = (cid * NS + sid) * per_tile + s * SG
            row0 = c0 * CHUNK
            pltpu.sync_copy(dst_hbm.at[pl.ds(c0, SG)], dst_v)
            pltpu.make_async_copy(
                hc_hbm.at[pl.ds(row0, CHUNK)], buf0, sem0).start()
            pltpu.make_async_copy(
                hc_hbm.at[pl.ds(row0 + CHUNK, CHUNK)], buf1, sem1).start()

            def _pair(g, carry):
                for b, (buf, sem) in enumerate(bufs):
                    ci = 2 * g + b
                    pltpu.make_async_copy(
                        hc_hbm.at[pl.ds(row0 + ci * CHUNK, CHUNK)],
                        buf, sem).wait()
                    pltpu.sync_copy(buf, acc_sh.at[dst_v.at[ci]], add=True)

                    @pl.when(ci + 2 < SG)
                    def _():
                        pltpu.make_async_copy(
                            hc_hbm.at[pl.ds(row0 + (ci + 2) * CHUNK, CHUNK)],
                            buf, sem).start()
                return carry
            lax.fori_loop(0, SG // 2, _pair, 0)

        _sc_epilogue(cid, sid, buf0, acc_sh, out_hbm, stripe)

    return body(hc, dst2)


def kernel(x, edge_index, edge_color, W1v, b1v, W2v, b2v, W1c, b1c, W2c, b2c):
    n, d = x.shape
    e = edge_index.shape[1]

    src = edge_index[0]
    dst = edge_index[1]

    hv = _mlp_v(x, W1v, b1v, W2v, b2v, blk=1000)

    epw = NS * CHUNK * SG
    e_pad = ((e + epw - 1) // epw) * epw

    hc = _mlp_c(edge_color.T, W1c, b1c, W2c, b2c, blk=1280, out_rows=e_pad)

    pad = e_pad - e
    src2 = jnp.concatenate(
        [src, jnp.zeros((pad,), jnp.int32)]).reshape(e_pad // CHUNK, CHUNK)
    dst2 = jnp.concatenate(
        [dst, jnp.full((pad,), n, jnp.int32)]).reshape(e_pad // CHUNK, CHUNK)

    n_acc = ((n + NS * CHUNK - 1) // (NS * CHUNK)) * (NS * CHUNK)
    pv = _sc_gather_scatter(hv, src2, dst2, n_acc)
    pc = _sc_linear_scatter(hc, dst2, n_acc)

    return _combine(pv, pc, n, blk=1000)


# speedup vs baseline: 1.3199x; 1.3199x over previous
import functools

import jax
import jax.numpy as jnp
from jax import lax
from jax.experimental import pallas as pl
from jax.experimental.pallas import tpu as pltpu
from jax.experimental.pallas import tpu_sc as plsc

NC = 2
NS = 16
LANES = 16
CHUNK = 128
SG = 40


def _mlp_v_body(x_ref, w1_ref, b1_ref, w2_ref, b2_ref, o_ref):
    h = jnp.maximum(
        jnp.dot(x_ref[...], w1_ref[...], preferred_element_type=jnp.float32)
        + b1_ref[...], 0.0)
    o_ref[...] = (
        jnp.dot(h, w2_ref[...], preferred_element_type=jnp.float32)
        + b2_ref[...])


def _mlp_v_body_rep(x_ref, w1_ref, b1_ref, w2_ref, b2_ref, o_ref):
    h = jnp.maximum(
        jnp.dot(x_ref[...], w1_ref[...], preferred_element_type=jnp.float32)
        + b1_ref[...], 0.0)
    o_ref[...] = (
        jnp.dot(h, w2_ref[...], preferred_element_type=jnp.float32)
        + b2_ref[...])[None]


def _mlp_v(xx, w1, b1, w2, b2, blk, k_rep):
    rows, din = xx.shape
    dh = w1.shape[1]
    dout = w2.shape[1]
    return pl.pallas_call(
        _mlp_v_body_rep,
        grid=(k_rep, rows // blk),
        in_specs=[
            pl.BlockSpec((blk, din), lambda k, i: (i, 0)),
            pl.BlockSpec((din, dh), lambda k, i: (0, 0)),
            pl.BlockSpec((1, dh), lambda k, i: (0, 0)),
            pl.BlockSpec((dh, dout), lambda k, i: (0, 0)),
            pl.BlockSpec((1, dout), lambda k, i: (0, 0)),
        ],
        out_specs=pl.BlockSpec((1, blk, dout), lambda k, i: (k, i, 0)),
        out_shape=jax.ShapeDtypeStruct((k_rep, rows, dout), jnp.float32),
    )(xx, w1, b1.reshape(1, dh), w2, b2.reshape(1, dout))


def _mlp_c_body(xt_ref, w1_ref, b1_ref, w2_ref, b2_ref, o_ref):
    h = jnp.maximum(
        lax.dot_general(xt_ref[...], w1_ref[...],
                        (((0,), (0,)), ((), ())),
                        preferred_element_type=jnp.float32)
        + b1_ref[...], 0.0)
    o_ref[...] = (
        jnp.dot(h.astype(jnp.bfloat16), w2_ref[...],
                preferred_element_type=jnp.float32)
        + b2_ref[...])


def _mlp_c(xt, w1, b1, w2, b2, blk, out_rows):
    din, rows = xt.shape
    dh = w1.shape[1]
    dout = w2.shape[1]
    last_real = rows // blk - 1
    return pl.pallas_call(
        _mlp_c_body,
        grid=(out_rows // blk,),
        in_specs=[
            pl.BlockSpec((din, blk), lambda i: (0, jnp.minimum(i, last_real))),
            pl.BlockSpec((din, dh), lambda i: (0, 0)),
            pl.BlockSpec((1, dh), lambda i: (0, 0)),
            pl.BlockSpec((dh, dout), lambda i: (0, 0)),
            pl.BlockSpec((1, dout), lambda i: (0, 0)),
        ],
        out_specs=pl.BlockSpec((blk, dout), lambda i: (i, 0)),
        out_shape=jax.ShapeDtypeStruct((out_rows, dout), jnp.float32),
    )(xt, w1, b1.reshape(1, dh), w2.astype(jnp.bfloat16),
      b2.reshape(1, dout))


def _combine_body(a_ref, b_ref, c_ref, e_ref, o_ref):
    o_ref[...] = (a_ref[0] + b_ref[0]) + (c_ref[0] + e_ref[0])


def _combine(pv, pc, n, blk):
    _, n_acc, d = pv.shape
    return pl.pallas_call(
        _combine_body,
        grid=(n // blk,),
        in_specs=[
            pl.BlockSpec((1, blk, d), lambda i: (0, i, 0)),
            pl.BlockSpec((1, blk, d), lambda i: (1, i, 0)),
            pl.BlockSpec((1, blk, d), lambda i: (0, i, 0)),
            pl.BlockSpec((1, blk, d), lambda i: (1, i, 0)),
        ],
        out_specs=pl.BlockSpec((blk, d), lambda i: (i, 0)),
        out_shape=jax.ShapeDtypeStruct((n, d), jnp.float32),
    )(pv, pv, pc, pc)


def _sc_prologue(sid, buf0, acc_sh, stripe, d):
    def _zrow(r, carry):
        for j in range(d // LANES):
            buf0[r, pl.ds(j * LANES, LANES)] = jnp.zeros((LANES,), jnp.float32)
        return carry
    lax.fori_loop(0, CHUNK, _zrow, 0)
    for k in range(stripe // CHUNK):
        pltpu.sync_copy(buf0,
                        acc_sh.at[pl.ds(sid * stripe + k * CHUNK, CHUNK)])
    plsc.subcore_barrier()


def _sc_epilogue(cid, sid, buf0, acc_sh, out_hbm, stripe):
    plsc.subcore_barrier()
    for k in range(stripe // CHUNK):
        r0 = sid * stripe + k * CHUNK
        pltpu.sync_copy(acc_sh.at[pl.ds(r0, CHUNK)], buf0)
        pltpu.sync_copy(buf0, out_hbm.at[cid, pl.ds(r0, CHUNK)])


V_SPLIT = (80, 80)
K_REP = 4


def _sc_gather_scatter(hv, src2, dst2, n_acc):
    n_nodes, d = hv.shape
    stripe = n_acc // NS

    mesh = plsc.VectorSubcoreMesh(core_axis_name="c", subcore_axis_name="s")

    @functools.partial(
        pl.kernel,
        mesh=mesh,
        out_type=jax.ShapeDtypeStruct((NC, n_acc, d), jnp.float32),
        scratch_types=[
            pltpu.VMEM((SG, CHUNK), jnp.int32),
            pltpu.VMEM((SG, CHUNK), jnp.int32),
            pltpu.VMEM((CHUNK, d), jnp.float32),
            pltpu.VMEM((CHUNK, d), jnp.float32),
            pltpu.VMEM_SHARED((n_acc, d), jnp.float32),
            pltpu.SemaphoreType.DMA,
            pltpu.SemaphoreType.DMA,
        ],
    )
    def body(hv_hbm, src_hbm, dst_hbm, out_hbm,
             src_v, dst_v, buf0, buf1, acc_sh, sem0, sem1):
        cid = lax.axis_index("c")
        sid = lax.axis_index("s")
        bufs = ((buf0, sem0), (buf1, sem1))
        _sc_prologue(sid, buf0, acc_sh, stripe, d)

        def _run(base, count):
            stage_sizes = [SG] * (count // SG)
            if count % SG:
                stage_sizes.append(count % SG)
            off = 0
            for sg in stage_sizes:
                c0 = base + off
                off += sg
                pltpu.sync_copy(src_hbm.at[pl.ds(c0, sg)],
                                src_v.at[pl.ds(0, sg)])
                pltpu.sync_copy(dst_hbm.at[pl.ds(c0, sg)],
                                dst_v.at[pl.ds(0, sg)])
                pltpu.make_async_copy(
                    hv_hbm.at[src_v.at[0]], buf0, sem0).start()
                pltpu.make_async_copy(
                    hv_hbm.at[src_v.at[1]], buf1, sem1).start()

                def _pair(g, carry):
                    for b, (buf, sem) in enumerate(bufs):
                        ci = 2 * g + b
                        pltpu.make_async_copy(
                            hv_hbm.at[src_v.at[ci]], buf, sem).wait()
                        pltpu.sync_copy(buf, acc_sh.at[dst_v.at[ci]],
                                        add=True)

                        @pl.when(ci + 2 < sg)
                        def _():
                            pltpu.make_async_copy(
                                hv_hbm.at[src_v.at[ci + 2]], buf, sem).start()
                    return carry
                lax.fori_loop(0, sg // 2, _pair, 0)

        if V_SPLIT[0]:
            @pl.when(cid == 0)
            def _core0():
                _run(sid * V_SPLIT[0], V_SPLIT[0])

        if V_SPLIT[1]:
            @pl.when(cid == 1)
            def _core1():
                _run(NS * V_SPLIT[0] + sid * V_SPLIT[1], V_SPLIT[1])

        _sc_epilogue(cid, sid, buf0, acc_sh, out_hbm, stripe)

    return body(hv, src2, dst2)


def _sc_linear_scatter(hc, dst2, n_acc):
    d = hc.shape[1]
    per_tile = dst2.shape[0] // (NC * NS)
    n_stages = per_tile // SG
    stripe = n_acc // NS

    mesh = plsc.VectorSubcoreMesh(core_axis_name="c", subcore_axis_name="s")

    @functools.partial(
        pl.kernel,
        mesh=mesh,
        out_type=jax.ShapeDtypeStruct((NC, n_acc, d), jnp.float32),
        scratch_types=[
            pltpu.VMEM((SG, CHUNK), jnp.int32),
            pltpu.VMEM((CHUNK, d), jnp.float32),
            pltpu.VMEM((CHUNK, d), jnp.float32),
            pltpu.VMEM_SHARED((n_acc, d), jnp.float32),
            pltpu.SemaphoreType.DMA,
            pltpu.SemaphoreType.DMA,
        ],
    )
    def body(hc_hbm, dst_hbm, out_hbm,
             dst_v, buf0, buf1, acc_sh, sem0, sem1):
        cid = lax.axis_index("c")
        sid = lax.axis_index("s")
        bufs = ((buf0, sem0), (buf1, sem1))
        _sc_prologue(sid, buf0, acc_sh, stripe, d)

        for s in range(n_stages):
            c0 = (cid * NS + sid) * per_tile + s * SG
            row0 = c0 * CHUNK
            pltpu.sync_copy(dst_hbm.at[pl.ds(c0, SG)], dst_v)
            pltpu.make_async_copy(
                hc_hbm.at[pl.ds(row0, CHUNK)], buf0, sem0).start()
            pltpu.make_async_copy(
                hc_hbm.at[pl.ds(row0 + CHUNK, CHUNK)], buf1, sem1).start()

            def _pair(g, carry):
                for b, (buf, sem) in enumerate(bufs):
                    ci = 2 * g + b
                    pltpu.make_async_copy(
                        hc_hbm.at[pl.ds(row0 + ci * CHUNK, CHUNK)],
                        buf, sem).wait()
                    pltpu.sync_copy(buf, acc_sh.at[dst_v.at[ci]], add=True)

                    @pl.when(ci + 2 < SG)
                    def _():
                        pltpu.make_async_copy(
                            hc_hbm.at[pl.ds(row0 + (ci + 2) * CHUNK, CHUNK)],
                            buf, sem).start()
                return carry
            lax.fori_loop(0, SG // 2, _pair, 0)

        _sc_epilogue(cid, sid, buf0, acc_sh, out_hbm, stripe)

    return body(hc, dst2)


def kernel(x, edge_index, edge_color, W1v, b1v, W2v, b2v, W1c, b1c, W2c, b2c):
    n, d = x.shape
    e = edge_index.shape[1]

    src = edge_index[0]
    dst = edge_index[1]

    hv = _mlp_v(x, W1v, b1v, W2v, b2v, blk=1000,
                k_rep=K_REP).reshape(K_REP * n, d)

    epw = NS * CHUNK * SG
    e_pad = ((e + epw - 1) // epw) * epw

    hc = _mlp_c(edge_color.T, W1c, b1c, W2c, b2c, blk=1280, out_rows=e_pad)

    pad = e_pad - e
    n_ch = e_pad // CHUNK
    src2 = jnp.concatenate(
        [src, jnp.zeros((pad,), jnp.int32)]).reshape(n_ch, CHUNK)
    src2 = src2 + ((jnp.arange(n_ch, dtype=jnp.int32) % K_REP) * n)[:, None]
    dst2 = jnp.concatenate(
        [dst, jnp.full((pad,), n, jnp.int32)]).reshape(e_pad // CHUNK, CHUNK)

    n_acc = ((n + NS * CHUNK - 1) // (NS * CHUNK)) * (NS * CHUNK)
    pv = _sc_gather_scatter(hv, src2, dst2, n_acc)
    pc = _sc_linear_scatter(hc, dst2, n_acc)

    return _combine(pv, pc, n, blk=1000)


# speedup vs baseline: 1.5487x; 1.1733x over previous
import functools

import jax
import jax.numpy as jnp
from jax import lax
from jax.experimental import pallas as pl
from jax.experimental.pallas import tpu as pltpu
from jax.experimental.pallas import tpu_sc as plsc

NC = 2
NS = 16
LANES = 16
CHUNK = 128
SG = 40


def _mlp_v_body(x_ref, w1_ref, b1_ref, w2_ref, b2_ref, o_ref):
    h = jnp.maximum(
        jnp.dot(x_ref[...], w1_ref[...], preferred_element_type=jnp.float32)
        + b1_ref[...], 0.0)
    o_ref[...] = (
        jnp.dot(h, w2_ref[...], preferred_element_type=jnp.float32)
        + b2_ref[...])


def _mlp_v_body_rep(x_ref, w1_ref, b1_ref, w2_ref, b2_ref, o_ref):
    h = jnp.maximum(
        jnp.dot(x_ref[...], w1_ref[...], preferred_element_type=jnp.float32)
        + b1_ref[...], 0.0)
    o_ref[...] = (
        jnp.dot(h, w2_ref[...], preferred_element_type=jnp.float32)
        + b2_ref[...])[None]


def _mlp_v(xx, w1, b1, w2, b2, blk, k_rep):
    rows, din = xx.shape
    dh = w1.shape[1]
    dout = w2.shape[1]
    return pl.pallas_call(
        _mlp_v_body_rep,
        grid=(k_rep, rows // blk),
        in_specs=[
            pl.BlockSpec((blk, din), lambda k, i: (i, 0)),
            pl.BlockSpec((din, dh), lambda k, i: (0, 0)),
            pl.BlockSpec((1, dh), lambda k, i: (0, 0)),
            pl.BlockSpec((dh, dout), lambda k, i: (0, 0)),
            pl.BlockSpec((1, dout), lambda k, i: (0, 0)),
        ],
        out_specs=pl.BlockSpec((1, blk, dout), lambda k, i: (k, i, 0)),
        out_shape=jax.ShapeDtypeStruct((k_rep, rows, dout), jnp.float32),
    )(xx, w1, b1.reshape(1, dh), w2, b2.reshape(1, dout))


def _mlp_c_body(xt_ref, w1_ref, b1_ref, w2_ref, b2_ref, o_ref):
    h = jnp.maximum(
        lax.dot_general(xt_ref[...].astype(jnp.bfloat16),
                        w1_ref[...].astype(jnp.bfloat16),
                        (((0,), (0,)), ((), ())),
                        preferred_element_type=jnp.float32)
        + b1_ref[...], 0.0)
    o_ref[...] = (
        jnp.dot(h.astype(jnp.bfloat16), w2_ref[...],
                preferred_element_type=jnp.float32)
        + b2_ref[...])


def _mlp_c(xt, w1, b1, w2, b2, blk, out_rows):
    din, rows = xt.shape
    dh = w1.shape[1]
    dout = w2.shape[1]
    last_real = rows // blk - 1
    return pl.pallas_call(
        _mlp_c_body,
        grid=(out_rows // blk,),
        in_specs=[
            pl.BlockSpec((din, blk), lambda i: (0, jnp.minimum(i, last_real))),
            pl.BlockSpec((din, dh), lambda i: (0, 0)),
            pl.BlockSpec((1, dh), lambda i: (0, 0)),
            pl.BlockSpec((dh, dout), lambda i: (0, 0)),
            pl.BlockSpec((1, dout), lambda i: (0, 0)),
        ],
        out_specs=pl.BlockSpec((blk, dout), lambda i: (i, 0)),
        out_shape=jax.ShapeDtypeStruct((out_rows, dout), jnp.float32),
    )(xt, w1, b1.reshape(1, dh), w2.astype(jnp.bfloat16),
      b2.reshape(1, dout))


def _combine_body(a_ref, b_ref, c_ref, e_ref, o_ref):
    o_ref[...] = (a_ref[0] + b_ref[0]) + (c_ref[0] + e_ref[0])


def _combine(pv, pc, n, blk):
    _, n_acc, d = pv.shape
    return pl.pallas_call(
        _combine_body,
        grid=(n // blk,),
        in_specs=[
            pl.BlockSpec((1, blk, d), lambda i: (0, i, 0)),
            pl.BlockSpec((1, blk, d), lambda i: (1, i, 0)),
            pl.BlockSpec((1, blk, d), lambda i: (0, i, 0)),
            pl.BlockSpec((1, blk, d), lambda i: (1, i, 0)),
        ],
        out_specs=pl.BlockSpec((blk, d), lambda i: (i, 0)),
        out_shape=jax.ShapeDtypeStruct((n, d), jnp.float32),
    )(pv, pv, pc, pc)


def _sc_prologue(sid, buf0, acc_sh, stripe, d):
    def _zrow(r, carry):
        for j in range(d // LANES):
            buf0[r, pl.ds(j * LANES, LANES)] = jnp.zeros((LANES,), jnp.float32)
        return carry
    lax.fori_loop(0, CHUNK, _zrow, 0)
    for k in range(stripe // CHUNK):
        pltpu.sync_copy(buf0,
                        acc_sh.at[pl.ds(sid * stripe + k * CHUNK, CHUNK)])
    plsc.subcore_barrier()


def _sc_epilogue(cid, sid, buf0, acc_sh, out_hbm, stripe):
    plsc.subcore_barrier()
    for k in range(stripe // CHUNK):
        r0 = sid * stripe + k * CHUNK
        pltpu.sync_copy(acc_sh.at[pl.ds(r0, CHUNK)], buf0)
        pltpu.sync_copy(buf0, out_hbm.at[cid, pl.ds(r0, CHUNK)])


V_SPLIT = (120, 40)
K_REP = 4


def _sc_gather_scatter(hv, src2, dst2, n_acc):
    n_nodes, d = hv.shape
    stripe = n_acc // NS

    mesh = plsc.VectorSubcoreMesh(core_axis_name="c", subcore_axis_name="s")

    @functools.partial(
        pl.kernel,
        mesh=mesh,
        out_type=jax.ShapeDtypeStruct((NC, n_acc, d), jnp.float32),
        scratch_types=[
            pltpu.VMEM((SG, CHUNK), jnp.int32),
            pltpu.VMEM((SG, CHUNK), jnp.int32),
            pltpu.VMEM((CHUNK, d), jnp.float32),
            pltpu.VMEM((CHUNK, d), jnp.float32),
            pltpu.VMEM_SHARED((n_acc, d), jnp.float32),
            pltpu.SemaphoreType.DMA,
            pltpu.SemaphoreType.DMA,
        ],
    )
    def body(hv_hbm, src_hbm, dst_hbm, out_hbm,
             src_v, dst_v, buf0, buf1, acc_sh, sem0, sem1):
        cid = lax.axis_index("c")
        sid = lax.axis_index("s")
        bufs = ((buf0, sem0), (buf1, sem1))
        _sc_prologue(sid, buf0, acc_sh, stripe, d)

        def _run(base, count):
            stage_sizes = [SG] * (count // SG)
            if count % SG:
                stage_sizes.append(count % SG)
            off = 0
            for sg in stage_sizes:
                c0 = base + off
                off += sg
                pltpu.sync_copy(src_hbm.at[pl.ds(c0, sg)],
                                src_v.at[pl.ds(0, sg)])
                pltpu.sync_copy(dst_hbm.at[pl.ds(c0, sg)],
                                dst_v.at[pl.ds(0, sg)])
                pltpu.make_async_copy(
                    hv_hbm.at[src_v.at[0]], buf0, sem0).start()
                pltpu.make_async_copy(
                    hv_hbm.at[src_v.at[1]], buf1, sem1).start()

                def _pair(g, carry):
                    for b, (buf, sem) in enumerate(bufs):
                        ci = 2 * g + b
                        pltpu.make_async_copy(
                            hv_hbm.at[src_v.at[ci]], buf, sem).wait()
                        pltpu.sync_copy(buf, acc_sh.at[dst_v.at[ci]],
                                        add=True)

                        @pl.when(ci + 2 < sg)
                        def _():
                            pltpu.make_async_copy(
                                hv_hbm.at[src_v.at[ci + 2]], buf, sem).start()
                    return carry
                lax.fori_loop(0, sg // 2, _pair, 0)

        if V_SPLIT[0]:
            @pl.when(cid == 0)
            def _core0():
                _run(sid * V_SPLIT[0], V_SPLIT[0])

        if V_SPLIT[1]:
            @pl.when(cid == 1)
            def _core1():
                _run(NS * V_SPLIT[0] + sid * V_SPLIT[1], V_SPLIT[1])

        _sc_epilogue(cid, sid, buf0, acc_sh, out_hbm, stripe)

    return body(hv, src2, dst2)


def _sc_linear_scatter(hc, dst2, n_acc):
    d = hc.shape[1]
    per_tile = dst2.shape[0] // (NC * NS)
    n_stages = per_tile // SG
    stripe = n_acc // NS

    mesh = plsc.VectorSubcoreMesh(core_axis_name="c", subcore_axis_name="s")

    @functools.partial(
        pl.kernel,
        mesh=mesh,
        out_type=jax.ShapeDtypeStruct((NC, n_acc, d), jnp.float32),
        scratch_types=[
            pltpu.VMEM((SG, CHUNK), jnp.int32),
            pltpu.VMEM((CHUNK, d), jnp.float32),
            pltpu.VMEM((CHUNK, d), jnp.float32),
            pltpu.VMEM_SHARED((n_acc, d), jnp.float32),
            pltpu.SemaphoreType.DMA,
            pltpu.SemaphoreType.DMA,
        ],
    )
    def body(hc_hbm, dst_hbm, out_hbm,
             dst_v, buf0, buf1, acc_sh, sem0, sem1):
        cid = lax.axis_index("c")
        sid = lax.axis_index("s")
        bufs = ((buf0, sem0), (buf1, sem1))
        _sc_prologue(sid, buf0, acc_sh, stripe, d)

        for s in range(n_stages):
            c0 = (cid * NS + sid) * per_tile + s * SG
            row0 = c0 * CHUNK
            pltpu.sync_copy(dst_hbm.at[pl.ds(c0, SG)], dst_v)
            pltpu.make_async_copy(
                hc_hbm.at[pl.ds(row0, CHUNK)], buf0, sem0).start()
            pltpu.make_async_copy(
                hc_hbm.at[pl.ds(row0 + CHUNK, CHUNK)], buf1, sem1).start()

            def _pair(g, carry):
                for b, (buf, sem) in enumerate(bufs):
                    ci = 2 * g + b
                    pltpu.make_async_copy(
                        hc_hbm.at[pl.ds(row0 + ci * CHUNK, CHUNK)],
                        buf, sem).wait()
                    pltpu.sync_copy(buf, acc_sh.at[dst_v.at[ci]], add=True)

                    @pl.when(ci + 2 < SG)
                    def _():
                        pltpu.make_async_copy(
                            hc_hbm.at[pl.ds(row0 + (ci + 2) * CHUNK, CHUNK)],
                            buf, sem).start()
                return carry
            lax.fori_loop(0, SG // 2, _pair, 0)

        _sc_epilogue(cid, sid, buf0, acc_sh, out_hbm, stripe)

    return body(hc, dst2)


def kernel(x, edge_index, edge_color, W1v, b1v, W2v, b2v, W1c, b1c, W2c, b2c):
    n, d = x.shape
    e = edge_index.shape[1]

    src = edge_index[0]
    dst = edge_index[1]

    hv = _mlp_v(x, W1v, b1v, W2v, b2v, blk=1000,
                k_rep=K_REP).reshape(K_REP * n, d)

    epw = NS * CHUNK * SG
    e_pad = ((e + epw - 1) // epw) * epw

    hc = _mlp_c(edge_color.T, W1c, b1c, W2c, b2c, blk=1280, out_rows=e_pad)

    pad = e_pad - e
    n_ch = e_pad // CHUNK
    src2 = jnp.concatenate(
        [src, jnp.zeros((pad,), jnp.int32)]).reshape(n_ch, CHUNK)
    src2 = src2 + ((jnp.arange(n_ch, dtype=jnp.int32) % K_REP) * n)[:, None]
    dst2 = jnp.concatenate(
        [dst, jnp.full((pad,), n, jnp.int32)]).reshape(e_pad // CHUNK, CHUNK)

    n_acc = ((n + NS * CHUNK - 1) // (NS * CHUNK)) * (NS * CHUNK)
    pv = _sc_gather_scatter(hv, src2, dst2, n_acc)
    pc = _sc_linear_scatter(hc, dst2, n_acc)

    return _combine(pv, pc, n, blk=1000)


# speedup vs baseline: 1.6248x; 1.0491x over previous
import functools

import jax
import jax.numpy as jnp
from jax import lax
from jax.experimental import pallas as pl
from jax.experimental.pallas import tpu as pltpu
from jax.experimental.pallas import tpu_sc as plsc

NC = 2
NS = 16
LANES = 16
CHUNK = 128
SG = 40


def _mlp_v_body(x_ref, w1_ref, b1_ref, w2_ref, b2_ref, o_ref):
    h = jnp.maximum(
        jnp.dot(x_ref[...], w1_ref[...], preferred_element_type=jnp.float32)
        + b1_ref[...], 0.0)
    o_ref[...] = (
        jnp.dot(h, w2_ref[...], preferred_element_type=jnp.float32)
        + b2_ref[...])


def _mlp_v_body_rep(x_ref, w1_ref, b1_ref, w2_ref, b2_ref, o_ref):
    h = jnp.maximum(
        jnp.dot(x_ref[...], w1_ref[...], preferred_element_type=jnp.float32)
        + b1_ref[...], 0.0)
    o_ref[...] = (
        jnp.dot(h, w2_ref[...], preferred_element_type=jnp.float32)
        + b2_ref[...])[None]


def _mlp_v(xx, w1, b1, w2, b2, blk, k_rep):
    rows, din = xx.shape
    dh = w1.shape[1]
    dout = w2.shape[1]
    return pl.pallas_call(
        _mlp_v_body_rep,
        grid=(k_rep, rows // blk),
        in_specs=[
            pl.BlockSpec((blk, din), lambda k, i: (i, 0)),
            pl.BlockSpec((din, dh), lambda k, i: (0, 0)),
            pl.BlockSpec((1, dh), lambda k, i: (0, 0)),
            pl.BlockSpec((dh, dout), lambda k, i: (0, 0)),
            pl.BlockSpec((1, dout), lambda k, i: (0, 0)),
        ],
        out_specs=pl.BlockSpec((1, blk, dout), lambda k, i: (k, i, 0)),
        out_shape=jax.ShapeDtypeStruct((k_rep, rows, dout), jnp.float32),
    )(xx, w1, b1.reshape(1, dh), w2, b2.reshape(1, dout))


def _mlp_c_body(xt_ref, w1_ref, b1_ref, w2_ref, b2_ref, o_ref):
    h = jnp.maximum(
        lax.dot_general(xt_ref[...].astype(jnp.bfloat16),
                        w1_ref[...].astype(jnp.bfloat16),
                        (((0,), (0,)), ((), ())),
                        preferred_element_type=jnp.float32)
        + b1_ref[...], 0.0)
    o_ref[...] = (
        jnp.dot(h.astype(jnp.bfloat16), w2_ref[...],
                preferred_element_type=jnp.float32)
        + b2_ref[...])


def _mlp_c(xt, w1, b1, w2, b2, blk, out_rows):
    din, rows = xt.shape
    dh = w1.shape[1]
    dout = w2.shape[1]
    last_real = rows // blk - 1
    return pl.pallas_call(
        _mlp_c_body,
        grid=(out_rows // blk,),
        in_specs=[
            pl.BlockSpec((din, blk), lambda i: (0, jnp.minimum(i, last_real))),
            pl.BlockSpec((din, dh), lambda i: (0, 0)),
            pl.BlockSpec((1, dh), lambda i: (0, 0)),
            pl.BlockSpec((dh, dout), lambda i: (0, 0)),
            pl.BlockSpec((1, dout), lambda i: (0, 0)),
        ],
        out_specs=pl.BlockSpec((blk, dout), lambda i: (i, 0)),
        out_shape=jax.ShapeDtypeStruct((out_rows, dout), jnp.float32),
    )(xt, w1, b1.reshape(1, dh), w2.astype(jnp.bfloat16),
      b2.reshape(1, dout))


def _combine_body(a_ref, b_ref, c_ref, e_ref, o_ref):
    o_ref[...] = (a_ref[0] + b_ref[0]) + (c_ref[0] + e_ref[0])


def _combine(pv, pc, n, blk):
    _, n_acc, d = pv.shape
    return pl.pallas_call(
        _combine_body,
        grid=(n // blk,),
        in_specs=[
            pl.BlockSpec((1, blk, d), lambda i: (0, i, 0)),
            pl.BlockSpec((1, blk, d), lambda i: (1, i, 0)),
            pl.BlockSpec((1, blk, d), lambda i: (0, i, 0)),
            pl.BlockSpec((1, blk, d), lambda i: (1, i, 0)),
        ],
        out_specs=pl.BlockSpec((blk, d), lambda i: (i, 0)),
        out_shape=jax.ShapeDtypeStruct((n, d), jnp.float32),
    )(pv, pv, pc, pc)


def _sc_prologue(sid, buf0, acc_sh, stripe, d):
    def _zrow(r, carry):
        for j in range(d // LANES):
            buf0[r, pl.ds(j * LANES, LANES)] = jnp.zeros((LANES,), jnp.float32)
        return carry
    lax.fori_loop(0, CHUNK, _zrow, 0)
    for k in range(stripe // CHUNK):
        pltpu.sync_copy(buf0,
                        acc_sh.at[pl.ds(sid * stripe + k * CHUNK, CHUNK)])
    plsc.subcore_barrier()


def _sc_epilogue(cid, sid, buf0, acc_sh, out_hbm, stripe):
    plsc.subcore_barrier()
    for k in range(stripe // CHUNK):
        r0 = sid * stripe + k * CHUNK
        pltpu.sync_copy(acc_sh.at[pl.ds(r0, CHUNK)], buf0)
        pltpu.sync_copy(buf0, out_hbm.at[cid, pl.ds(r0, CHUNK)])


V_SPLIT = (128, 32)
K_REP = 4


def _sc_gather_scatter(hv, src2, dst2, n_acc):
    n_nodes, d = hv.shape
    stripe = n_acc // NS

    mesh = plsc.VectorSubcoreMesh(core_axis_name="c", subcore_axis_name="s")

    @functools.partial(
        pl.kernel,
        mesh=mesh,
        out_type=jax.ShapeDtypeStruct((NC, n_acc, d), jnp.float32),
        scratch_types=[
            pltpu.VMEM((SG, CHUNK), jnp.int32),
            pltpu.VMEM((SG, CHUNK), jnp.int32),
            pltpu.VMEM((CHUNK, d), jnp.float32),
            pltpu.VMEM((CHUNK, d), jnp.float32),
            pltpu.VMEM_SHARED((n_acc, d), jnp.float32),
            pltpu.SemaphoreType.DMA,
            pltpu.SemaphoreType.DMA,
        ],
    )
    def body(hv_hbm, src_hbm, dst_hbm, out_hbm,
             src_v, dst_v, buf0, buf1, acc_sh, sem0, sem1):
        cid = lax.axis_index("c")
        sid = lax.axis_index("s")
        bufs = ((buf0, sem0), (buf1, sem1))
        _sc_prologue(sid, buf0, acc_sh, stripe, d)

        def _run(base, count):
            stage_sizes = [SG] * (count // SG)
            if count % SG:
                stage_sizes.append(count % SG)
            off = 0
            for sg in stage_sizes:
                c0 = base + off
                off += sg
                pltpu.sync_copy(src_hbm.at[pl.ds(c0, sg)],
                                src_v.at[pl.ds(0, sg)])
                pltpu.sync_copy(dst_hbm.at[pl.ds(c0, sg)],
                                dst_v.at[pl.ds(0, sg)])
                pltpu.make_async_copy(
                    hv_hbm.at[src_v.at[0]], buf0, sem0).start()
                pltpu.make_async_copy(
                    hv_hbm.at[src_v.at[1]], buf1, sem1).start()

                def _pair(g, carry):
                    for b, (buf, sem) in enumerate(bufs):
                        ci = 2 * g + b
                        pltpu.make_async_copy(
                            hv_hbm.at[src_v.at[ci]], buf, sem).wait()
                        pltpu.sync_copy(buf, acc_sh.at[dst_v.at[ci]],
                                        add=True)

                        @pl.when(ci + 2 < sg)
                        def _():
                            pltpu.make_async_copy(
                                hv_hbm.at[src_v.at[ci + 2]], buf, sem).start()
                    return carry
                lax.fori_loop(0, sg // 2, _pair, 0)

        if V_SPLIT[0]:
            @pl.when(cid == 0)
            def _core0():
                _run(sid * V_SPLIT[0], V_SPLIT[0])

        if V_SPLIT[1]:
            @pl.when(cid == 1)
            def _core1():
                _run(NS * V_SPLIT[0] + sid * V_SPLIT[1], V_SPLIT[1])

        _sc_epilogue(cid, sid, buf0, acc_sh, out_hbm, stripe)

    return body(hv, src2, dst2)


def _sc_linear_scatter(hc, dst2, n_acc):
    d = hc.shape[1]
    per_tile = dst2.shape[0] // (NC * NS)
    n_stages = per_tile // SG
    stripe = n_acc // NS

    mesh = plsc.VectorSubcoreMesh(core_axis_name="c", subcore_axis_name="s")

    @functools.partial(
        pl.kernel,
        mesh=mesh,
        out_type=jax.ShapeDtypeStruct((NC, n_acc, d), jnp.float32),
        scratch_types=[
            pltpu.VMEM((SG, CHUNK), jnp.int32),
            pltpu.VMEM((CHUNK, d), jnp.float32),
            pltpu.VMEM((CHUNK, d), jnp.float32),
            pltpu.VMEM_SHARED((n_acc, d), jnp.float32),
            pltpu.SemaphoreType.DMA,
            pltpu.SemaphoreType.DMA,
        ],
    )
    def body(hc_hbm, dst_hbm, out_hbm,
             dst_v, buf0, buf1, acc_sh, sem0, sem1):
        cid = lax.axis_index("c")
        sid = lax.axis_index("s")
        bufs = ((buf0, sem0), (buf1, sem1))
        _sc_prologue(sid, buf0, acc_sh, stripe, d)

        for s in range(n_stages):
            c0 = (cid * NS + sid) * per_tile + s * SG
            row0 = c0 * CHUNK
            pltpu.sync_copy(dst_hbm.at[pl.ds(c0, SG)], dst_v)
            pltpu.make_async_copy(
                hc_hbm.at[pl.ds(row0, CHUNK)], buf0, sem0).start()
            pltpu.make_async_copy(
                hc_hbm.at[pl.ds(row0 + CHUNK, CHUNK)], buf1, sem1).start()

            def _pair(g, carry):
                for b, (buf, sem) in enumerate(bufs):
                    ci = 2 * g + b
                    pltpu.make_async_copy(
                        hc_hbm.at[pl.ds(row0 + ci * CHUNK, CHUNK)],
                        buf, sem).wait()
                    pltpu.sync_copy(buf, acc_sh.at[dst_v.at[ci]], add=True)

                    @pl.when(ci + 2 < SG)
                    def _():
                        pltpu.make_async_copy(
                            hc_hbm.at[pl.ds(row0 + (ci + 2) * CHUNK, CHUNK)],
                            buf, sem).start()
                return carry
            lax.fori_loop(0, SG // 2, _pair, 0)

        _sc_epilogue(cid, sid, buf0, acc_sh, out_hbm, stripe)

    return body(hc, dst2)


def kernel(x, edge_index, edge_color, W1v, b1v, W2v, b2v, W1c, b1c, W2c, b2c):
    n, d = x.shape
    e = edge_index.shape[1]

    src = edge_index[0]
    dst = edge_index[1]

    hv = _mlp_v(x, W1v, b1v, W2v, b2v, blk=1000,
                k_rep=K_REP).reshape(K_REP * n, d)

    epw = NS * CHUNK * SG
    e_pad = ((e + epw - 1) // epw) * epw

    hc = _mlp_c(edge_color.T, W1c, b1c, W2c, b2c, blk=1280, out_rows=e_pad)

    pad = e_pad - e
    n_ch = e_pad // CHUNK
    src2 = jnp.concatenate(
        [src, jnp.zeros((pad,), jnp.int32)]).reshape(n_ch, CHUNK)
    src2 = src2 + ((jnp.arange(n_ch, dtype=jnp.int32) % K_REP) * n)[:, None]
    dst2 = jnp.concatenate(
        [dst, jnp.full((pad,), n, jnp.int32)]).reshape(e_pad // CHUNK, CHUNK)

    n_acc = ((n + NS * CHUNK - 1) // (NS * CHUNK)) * (NS * CHUNK)
    pv = _sc_gather_scatter(hv, src2, dst2, n_acc)
    pc = _sc_linear_scatter(hc, dst2, n_acc)

    return _combine(pv, pc, n, blk=1000)
